# Initial kernel scaffold; baseline (speedup 1.0000x reference)
import functools

import jax
import jax.numpy as jnp
from jax import lax
from jax.experimental import pallas as pl
from jax.experimental.pallas import tpu as pltpu
from jax.experimental.pallas import tpu_sc as plsc

N = 10000
E = 320000
B = 4
S = 2048
H = 1024
D = 128
KGE = 128
P = 32
NPAD = 10240
NTILES = 32
ROWS_PER_TILE = NPAD // NTILES
STRIPE = NPAD // 16
EPT = E // NTILES
EC = 80
NCHUNK = EPT // EC

_mesh = lambda: plsc.VectorSubcoreMesh(core_axis_name="c", subcore_axis_name="s")


def _emb_gather(emb, gx_pad):
    @functools.partial(
        pl.kernel,
        out_type=jax.ShapeDtypeStruct((NPAD, KGE), jnp.float32),
        mesh=_mesh(),
        scratch_types=[
            pltpu.VMEM((EC,), jnp.int32),
            pltpu.VMEM((EC, KGE), jnp.float32),
            pltpu.SemaphoreType.DMA,
        ],
    )
    def k(emb_hbm, idx_hbm, out_hbm, idx_v, rows_v, sem):
        cid = lax.axis_index("c")
        sid = lax.axis_index("s")
        tid = cid * 16 + sid
        base = tid * ROWS_PER_TILE
        for i in range(ROWS_PER_TILE // EC):
            off = pl.multiple_of(base + i * EC, 8)
            pltpu.sync_copy(idx_hbm.at[pl.ds(off, EC)], idx_v)
            pltpu.async_copy(emb_hbm.at[idx_v], rows_v, sem).wait()
            pltpu.sync_copy(rows_v, out_hbm.at[pl.ds(off, EC)])

    return k(emb, gx_pad)


def _edge_pass(h, asrc, adst, src, dst, zrows, zs):
    @functools.partial(
        pl.kernel,
        out_type=(
            jax.ShapeDtypeStruct((2, NPAD, KGE), jnp.float32),
            jax.ShapeDtypeStruct((2, NPAD), jnp.float32),
        ),
        mesh=_mesh(),
        scratch_types=[
            pltpu.VMEM((NPAD,), jnp.float32),
            pltpu.VMEM((NPAD,), jnp.float32),
            pltpu.VMEM((EC,), jnp.int32),
            pltpu.VMEM((EC,), jnp.int32),
            pltpu.VMEM((EC,), jnp.float32),
            pltpu.VMEM((EC, KGE), jnp.float32),
            pltpu.VMEM_SHARED((NPAD, KGE), jnp.float32),
            pltpu.VMEM_SHARED((NPAD,), jnp.float32),
            pltpu.SemaphoreType.DMA,
        ],
    )
    def k(h_hbm, asrc_hbm, adst_hbm, src_hbm, dst_hbm, zr_hbm, zs_hbm,
          u_out, s_out, asrc_v, adst_v, src_v, dst_v, ex_v, rows_v,
          u_sh, s_sh, sem):
        cid = lax.axis_index("c")
        sid = lax.axis_index("s")
        tid = cid * 16 + sid
        stripe = pl.multiple_of(sid * STRIPE, 8)
        pltpu.sync_copy(zr_hbm, u_sh.at[pl.ds(stripe, STRIPE)])
        pltpu.sync_copy(zs_hbm, s_sh.at[pl.ds(stripe, STRIPE)])
        pltpu.sync_copy(asrc_hbm, asrc_v)
        pltpu.sync_copy(adst_hbm, adst_v)
        plsc.subcore_barrier()

        ebase = tid * EPT

        def chunk(i, carry):
            off = pl.multiple_of(ebase + i * EC, 8)
            pltpu.sync_copy(src_hbm.at[pl.ds(off, EC)], src_v)
            pltpu.sync_copy(dst_hbm.at[pl.ds(off, EC)], dst_v)
            for g in range(EC // 16):
                sidx = src_v[pl.ds(g * 16, 16)]
                didx = dst_v[pl.ds(g * 16, 16)]
                a = plsc.load_gather(asrc_v, [sidx])
                b = plsc.load_gather(adst_v, [didx])
                x = a + b
                x = jnp.where(x > 0.0, x, 0.2 * x)
                ex_v[pl.ds(g * 16, 16)] = jnp.exp(x)
            pltpu.async_copy(h_hbm.at[src_v], rows_v, sem).wait()

            def row(j, c2):
                w = plsc.load_gather(ex_v, [jnp.full((16,), j, jnp.int32)])
                for k2 in range(KGE // 16):
                    rows_v[j, pl.ds(k2 * 16, 16)] = (
                        rows_v[j, pl.ds(k2 * 16, 16)] * w)
                return c2

            lax.fori_loop(0, EC, row, 0)
            pltpu.sync_copy(rows_v, u_sh.at[dst_v], add=True)
            pltpu.sync_copy(ex_v, s_sh.at[dst_v], add=True)
            return carry

        lax.fori_loop(0, NCHUNK, chunk, 0)
        plsc.subcore_barrier()
        pltpu.sync_copy(u_sh.at[pl.ds(stripe, STRIPE)],
                        u_out.at[cid, pl.ds(stripe, STRIPE)])
        pltpu.sync_copy(s_sh.at[pl.ds(stripe, STRIPE)],
                        s_out.at[cid, pl.ds(stripe, STRIPE)])

    return k(h, asrc, adst, src, dst, zrows, zs)


def _tc_pre(x, W, A):
    def body(x_ref, w_ref, a_ref, h_ref, av_ref):
        hv = jnp.dot(x_ref[...], w_ref[...], preferred_element_type=jnp.float32)
        h_ref[...] = hv
        av_ref[...] = jnp.dot(hv, a_ref[...], preferred_element_type=jnp.float32)

    return pl.pallas_call(
        body,
        out_shape=(
            jax.ShapeDtypeStruct((NPAD, KGE), jnp.float32),
            jax.ShapeDtypeStruct((NPAD, KGE), jnp.float32),
        ),
    )(x, W, A)


def _tc_mid(u, s3, h, av, b_row, W2, A2):
    def body(u_ref, s_ref, h_ref, av_ref, b_ref, w_ref, a_ref, h2_ref, av2_ref):
        exl = jnp.exp(jnp.where(av_ref[:, 0:1] > 0.0, av_ref[:, 0:1],
                                0.2 * av_ref[:, 0:1]))
        stot = s_ref[0] + s_ref[1] + exl + 1e-16
        x2 = (u_ref[0] + u_ref[1] + exl * h_ref[...]) / stot + b_ref[...]
        x2 = jnp.maximum(x2, 0.0)
        h2 = jnp.dot(x2, w_ref[...], preferred_element_type=jnp.float32)
        h2_ref[...] = h2
        av2_ref[...] = jnp.dot(h2, a_ref[...], preferred_element_type=jnp.float32)

    return pl.pallas_call(
        body,
        out_shape=(
            jax.ShapeDtypeStruct((NPAD, KGE), jnp.float32),
            jax.ShapeDtypeStruct((NPAD, KGE), jnp.float32),
        ),
    )(u, s3, h, av, b_row, W2, A2)


def _tc_post(u, s3, h, av, b_row, Wc, bc_row):
    def body(u_ref, s_ref, h_ref, av_ref, b_ref, wc_ref, bc_ref, hc_ref):
        exl = jnp.exp(jnp.where(av_ref[:, 0:1] > 0.0, av_ref[:, 0:1],
                                0.2 * av_ref[:, 0:1]))
        stot = s_ref[0] + s_ref[1] + exl + 1e-16
        out2 = (u_ref[0] + u_ref[1] + exl * h_ref[...]) / stot + b_ref[...]
        hc_ref[...] = jnp.dot(out2, wc_ref[...],
                              preferred_element_type=jnp.float32) + bc_ref[...]

    return pl.pallas_call(
        body,
        out_shape=jax.ShapeDtypeStruct((NPAD, KGE), jnp.float32),
    )(u, s3, h, av, b_row, Wc, bc_row)


def _tc_pool(hc, batch2d, hidden, seq_lengths, Wd, bd_row, Wu, bu_row):
    def body(hc_ref, b2d_ref, hid_ref, sl_ref, wd_ref, bd_ref, wu_ref,
             bu_ref, out_ref):
        i = pl.program_id(0)
        b2d = b2d_ref[...]
        c = jnp.sum(jnp.where(b2d == i, 1, 0))
        off = jnp.sum(jnp.where(b2d < i, 1, 0))
        sl = sl_ref[i]
        L = c + sl
        p2 = lax.broadcasted_iota(jnp.int32, (P, 1), 0)
        s_ = (p2 * L) // P
        e_ = ((p2 + 1) * L + (P - 1)) // P
        lo_n = off + s_
        hi_n = off + jnp.minimum(e_, c)
        iota_n = lax.broadcasted_iota(jnp.int32, (P, NPAD), 1)
        mask_n = ((iota_n >= lo_n) & (iota_n < hi_n)).astype(jnp.float32)
        s_hc = jnp.dot(mask_n, hc_ref[...], preferred_element_type=jnp.float32)
        lo_h = jnp.maximum(s_, c) - c
        hi_h = jnp.maximum(e_ - c, 0)
        iota_s = lax.broadcasted_iota(jnp.int32, (P, S), 1)
        mask_h = ((iota_s >= lo_h) & (iota_s < hi_h)).astype(jnp.float32)
        s_raw = jnp.dot(mask_h, hid_ref[0], preferred_element_type=jnp.float32)
        s_hid = jnp.dot(s_raw, wd_ref[...], preferred_element_type=jnp.float32)
        nh = (hi_h - lo_h).astype(jnp.float32)
        w = (e_ - s_).astype(jnp.float32)
        pool = (s_hc + s_hid + nh * bd_ref[...]) / w
        silu = pool * (1.0 / (1.0 + jnp.exp(-pool)))
        out_ref[0] = jnp.dot(silu, wu_ref[...],
                             preferred_element_type=jnp.float32) + bu_ref[...]

    return pl.pallas_call(
        body,
        grid=(B,),
        in_specs=[
            pl.BlockSpec((NPAD, KGE), lambda i: (0, 0)),
            pl.BlockSpec((NPAD // 128, 128), lambda i: (0, 0)),
            pl.BlockSpec((1, S, H), lambda i: (i, 0, 0)),
            pl.BlockSpec(memory_space=pltpu.SMEM),
            pl.BlockSpec((H, D), lambda i: (0, 0)),
            pl.BlockSpec((1, D), lambda i: (0, 0)),
            pl.BlockSpec((D, H), lambda i: (0, 0)),
            pl.BlockSpec((1, H), lambda i: (0, 0)),
        ],
        out_specs=pl.BlockSpec((1, P, H), lambda i: (i, 0, 0)),
        out_shape=jax.ShapeDtypeStruct((B, P, H), jnp.float32),
    )(hc, batch2d, hidden, seq_lengths, Wd, bd_row, Wu, bu_row)


def kernel(graph_x, edge_index, batch, hidden_states, seq_lengths, emb,
           W1, as1, ad1, b1, W2, as2, ad2, b2, Wd, bd, Wc, bc, Wu, bu):
    f32 = jnp.float32
    gx_pad = jnp.pad(graph_x, (0, NPAD - N))
    src = edge_index[0]
    dst = edge_index[1]
    batch2d = jnp.pad(batch, (0, NPAD - N), constant_values=127).reshape(
        NPAD // 128, 128)
    zrows = jnp.zeros((STRIPE, KGE), f32)
    zs = jnp.zeros((STRIPE,), f32)
    zcol = jnp.zeros((KGE, KGE - 3), f32)
    A1 = jnp.concatenate(
        [(as1 + ad1)[:, None], as1[:, None], ad1[:, None], zcol], axis=1)
    A2 = jnp.concatenate(
        [(as2 + ad2)[:, None], as2[:, None], ad2[:, None], zcol], axis=1)

    gx = _emb_gather(emb, gx_pad)

    h1, av1 = _tc_pre(gx, W1, A1)
    u1, s1 = _edge_pass(h1, av1[:, 1], av1[:, 2], src, dst, zrows, zs)
    h2, av2 = _tc_mid(u1, s1.reshape(2, NPAD, 1), h1, av1,
                      b1.reshape(1, KGE), W2, A2)
    u2, s2 = _edge_pass(h2, av2[:, 1], av2[:, 2], src, dst, zrows, zs)
    hc = _tc_post(u2, s2.reshape(2, NPAD, 1), h2, av2,
                  b2.reshape(1, KGE), Wc, bc.reshape(1, KGE))

    return _tc_pool(hc, batch2d, hidden_states, seq_lengths,
                    Wd, bd.reshape(1, D), Wu, bu.reshape(1, H))


# speedup vs baseline: 25.5052x; 25.5052x over previous
import functools

import jax
import jax.numpy as jnp
from jax import lax
from jax.experimental import pallas as pl
from jax.experimental.pallas import tpu as pltpu
from jax.experimental.pallas import tpu_sc as plsc

N = 10000
E = 320000
B = 4
S = 2048
H = 1024
D = 128
KGE = 128
P = 32
NPAD = 10240
NTILES = 32
ROWS_PER_TILE = NPAD // NTILES
STRIPE = NPAD // 16
EPT = E // NTILES
EC = 80
NCHUNK = EPT // EC

_mesh = lambda: plsc.VectorSubcoreMesh(core_axis_name="c", subcore_axis_name="s")


def _emb_gather(emb, gx_pad):
    @functools.partial(
        pl.kernel,
        out_type=jax.ShapeDtypeStruct((NPAD, KGE), jnp.float32),
        mesh=_mesh(),
        scratch_types=[
            pltpu.VMEM((EC,), jnp.int32),
            pltpu.VMEM((EC, KGE), jnp.float32),
            pltpu.SemaphoreType.DMA,
        ],
    )
    def k(emb_hbm, idx_hbm, out_hbm, idx_v, rows_v, sem):
        cid = lax.axis_index("c")
        sid = lax.axis_index("s")
        tid = cid * 16 + sid
        base = tid * ROWS_PER_TILE
        for i in range(ROWS_PER_TILE // EC):
            off = pl.multiple_of(base + i * EC, 8)
            pltpu.sync_copy(idx_hbm.at[pl.ds(off, EC)], idx_v)
            pltpu.async_copy(emb_hbm.at[idx_v], rows_v, sem).wait()
            pltpu.sync_copy(rows_v, out_hbm.at[pl.ds(off, EC)])

    return k(emb, gx_pad)


def _edge_pass(h, asrc, adst, src, dst, zrows, zs):
    @functools.partial(
        pl.kernel,
        out_type=(
            jax.ShapeDtypeStruct((2, NPAD, KGE), jnp.float32),
            jax.ShapeDtypeStruct((2, NPAD), jnp.float32),
        ),
        mesh=_mesh(),
        scratch_types=[
            pltpu.VMEM((EC,), jnp.int32),
            pltpu.VMEM((EC,), jnp.int32),
            pltpu.VMEM((EC,), jnp.float32),
            pltpu.VMEM((EC,), jnp.float32),
            pltpu.VMEM((EC,), jnp.float32),
            pltpu.VMEM((EC, KGE), jnp.float32),
            pltpu.VMEM_SHARED((NPAD, KGE), jnp.float32),
            pltpu.VMEM_SHARED((NPAD,), jnp.float32),
            pltpu.SemaphoreType.DMA,
            pltpu.SemaphoreType.DMA,
        ],
    )
    def k(h_hbm, asrc_hbm, adst_hbm, src_hbm, dst_hbm, zr_hbm, zs_hbm,
          u_out, s_out, src_v, dst_v, a_v, b_v, ex_v, rows_v,
          u_sh, s_sh, sem_s, sem_r):
        cid = lax.axis_index("c")
        sid = lax.axis_index("s")
        tid = cid * 16 + sid
        stripe = pl.multiple_of(sid * STRIPE, 8)
        pltpu.sync_copy(zr_hbm, u_sh.at[pl.ds(stripe, STRIPE)])
        pltpu.sync_copy(zs_hbm, s_sh.at[pl.ds(stripe, STRIPE)])
        plsc.subcore_barrier()

        ebase = tid * EPT

        def chunk(i, carry):
            off = pl.multiple_of(ebase + i * EC, 8)
            pltpu.sync_copy(src_hbm.at[pl.ds(off, EC)], src_v)
            pltpu.sync_copy(dst_hbm.at[pl.ds(off, EC)], dst_v)
            row_cp = pltpu.async_copy(h_hbm.at[src_v], rows_v, sem_r)
            a_cp = pltpu.async_copy(asrc_hbm.at[src_v], a_v, sem_s)
            b_cp = pltpu.async_copy(adst_hbm.at[dst_v], b_v, sem_s)
            a_cp.wait()
            b_cp.wait()
            for g in range(EC // 16):
                x = a_v[pl.ds(g * 16, 16)] + b_v[pl.ds(g * 16, 16)]
                x = jnp.where(x > 0.0, x, 0.2 * x)
                ex_v[pl.ds(g * 16, 16)] = jnp.exp(x)
            row_cp.wait()

            def rowgrp(g2, c2):
                exvec = ex_v[pl.ds(g2 * 16, 16)]
                for l in range(16):
                    w = jnp.full((16,), exvec[l], jnp.float32)
                    j = g2 * 16 + l
                    for k2 in range(KGE // 16):
                        rows_v[j, pl.ds(k2 * 16, 16)] = (
                            rows_v[j, pl.ds(k2 * 16, 16)] * w)
                return c2

            lax.fori_loop(0, EC // 16, rowgrp, 0)
            pltpu.sync_copy(rows_v, u_sh.at[dst_v], add=True)
            pltpu.sync_copy(ex_v, s_sh.at[dst_v], add=True)
            return carry

        lax.fori_loop(0, NCHUNK, chunk, 0)
        plsc.subcore_barrier()
        pltpu.sync_copy(u_sh.at[pl.ds(stripe, STRIPE)],
                        u_out.at[cid, pl.ds(stripe, STRIPE)])
        pltpu.sync_copy(s_sh.at[pl.ds(stripe, STRIPE)],
                        s_out.at[cid, pl.ds(stripe, STRIPE)])

    return k(h, asrc, adst, src, dst, zrows, zs)


def _tc_pre(x, W, A):
    def body(x_ref, w_ref, a_ref, h_ref, av_ref):
        hv = jnp.dot(x_ref[...], w_ref[...], preferred_element_type=jnp.float32)
        h_ref[...] = hv
        av_ref[...] = jnp.dot(hv, a_ref[...], preferred_element_type=jnp.float32)

    return pl.pallas_call(
        body,
        out_shape=(
            jax.ShapeDtypeStruct((NPAD, KGE), jnp.float32),
            jax.ShapeDtypeStruct((NPAD, KGE), jnp.float32),
        ),
    )(x, W, A)


def _tc_mid(u, s3, h, av, b_row, W2, A2):
    def body(u_ref, s_ref, h_ref, av_ref, b_ref, w_ref, a_ref, h2_ref, av2_ref):
        exl = jnp.exp(jnp.where(av_ref[:, 0:1] > 0.0, av_ref[:, 0:1],
                                0.2 * av_ref[:, 0:1]))
        stot = s_ref[0] + s_ref[1] + exl + 1e-16
        x2 = (u_ref[0] + u_ref[1] + exl * h_ref[...]) / stot + b_ref[...]
        x2 = jnp.maximum(x2, 0.0)
        h2 = jnp.dot(x2, w_ref[...], preferred_element_type=jnp.float32)
        h2_ref[...] = h2
        av2_ref[...] = jnp.dot(h2, a_ref[...], preferred_element_type=jnp.float32)

    return pl.pallas_call(
        body,
        out_shape=(
            jax.ShapeDtypeStruct((NPAD, KGE), jnp.float32),
            jax.ShapeDtypeStruct((NPAD, KGE), jnp.float32),
        ),
    )(u, s3, h, av, b_row, W2, A2)


def _tc_post(u, s3, h, av, b_row, Wc, bc_row):
    def body(u_ref, s_ref, h_ref, av_ref, b_ref, wc_ref, bc_ref, hc_ref):
        exl = jnp.exp(jnp.where(av_ref[:, 0:1] > 0.0, av_ref[:, 0:1],
                                0.2 * av_ref[:, 0:1]))
        stot = s_ref[0] + s_ref[1] + exl + 1e-16
        out2 = (u_ref[0] + u_ref[1] + exl * h_ref[...]) / stot + b_ref[...]
        hc_ref[...] = jnp.dot(out2, wc_ref[...],
                              preferred_element_type=jnp.float32) + bc_ref[...]

    return pl.pallas_call(
        body,
        out_shape=jax.ShapeDtypeStruct((NPAD, KGE), jnp.float32),
    )(u, s3, h, av, b_row, Wc, bc_row)


def _tc_pool(hc, batch2d, hidden, seq_lengths, Wd, bd_row, Wu, bu_row):
    def body(hc_ref, b2d_ref, hid_ref, sl_ref, wd_ref, bd_ref, wu_ref,
             bu_ref, out_ref):
        i = pl.program_id(0)
        b2d = b2d_ref[...]
        c = jnp.sum(jnp.where(b2d == i, 1, 0))
        off = jnp.sum(jnp.where(b2d < i, 1, 0))
        sl = sl_ref[i]
        L = c + sl
        p2 = lax.broadcasted_iota(jnp.int32, (P, 1), 0)
        s_ = (p2 * L) // P
        e_ = ((p2 + 1) * L + (P - 1)) // P
        lo_n = off + s_
        hi_n = off + jnp.minimum(e_, c)
        iota_n = lax.broadcasted_iota(jnp.int32, (P, NPAD), 1)
        mask_n = ((iota_n >= lo_n) & (iota_n < hi_n)).astype(jnp.float32)
        s_hc = jnp.dot(mask_n, hc_ref[...], preferred_element_type=jnp.float32)
        lo_h = jnp.maximum(s_, c) - c
        hi_h = jnp.maximum(e_ - c, 0)
        iota_s = lax.broadcasted_iota(jnp.int32, (P, S), 1)
        mask_h = ((iota_s >= lo_h) & (iota_s < hi_h)).astype(jnp.float32)
        s_raw = jnp.dot(mask_h, hid_ref[0], preferred_element_type=jnp.float32)
        s_hid = jnp.dot(s_raw, wd_ref[...], preferred_element_type=jnp.float32)
        nh = (hi_h - lo_h).astype(jnp.float32)
        w = (e_ - s_).astype(jnp.float32)
        pool = (s_hc + s_hid + nh * bd_ref[...]) / w
        silu = pool * (1.0 / (1.0 + jnp.exp(-pool)))
        out_ref[0] = jnp.dot(silu, wu_ref[...],
                             preferred_element_type=jnp.float32) + bu_ref[...]

    return pl.pallas_call(
        body,
        grid=(B,),
        in_specs=[
            pl.BlockSpec((NPAD, KGE), lambda i: (0, 0)),
            pl.BlockSpec((NPAD // 128, 128), lambda i: (0, 0)),
            pl.BlockSpec((1, S, H), lambda i: (i, 0, 0)),
            pl.BlockSpec(memory_space=pltpu.SMEM),
            pl.BlockSpec((H, D), lambda i: (0, 0)),
            pl.BlockSpec((1, D), lambda i: (0, 0)),
            pl.BlockSpec((D, H), lambda i: (0, 0)),
            pl.BlockSpec((1, H), lambda i: (0, 0)),
        ],
        out_specs=pl.BlockSpec((1, P, H), lambda i: (i, 0, 0)),
        out_shape=jax.ShapeDtypeStruct((B, P, H), jnp.float32),
    )(hc, batch2d, hidden, seq_lengths, Wd, bd_row, Wu, bu_row)


def kernel(graph_x, edge_index, batch, hidden_states, seq_lengths, emb,
           W1, as1, ad1, b1, W2, as2, ad2, b2, Wd, bd, Wc, bc, Wu, bu):
    f32 = jnp.float32
    gx_pad = jnp.pad(graph_x, (0, NPAD - N))
    src = edge_index[0]
    dst = edge_index[1]
    batch2d = jnp.pad(batch, (0, NPAD - N), constant_values=127).reshape(
        NPAD // 128, 128)
    zrows = jnp.zeros((STRIPE, KGE), f32)
    zs = jnp.zeros((STRIPE,), f32)
    zcol = jnp.zeros((KGE, KGE - 3), f32)
    A1 = jnp.concatenate(
        [(as1 + ad1)[:, None], as1[:, None], ad1[:, None], zcol], axis=1)
    A2 = jnp.concatenate(
        [(as2 + ad2)[:, None], as2[:, None], ad2[:, None], zcol], axis=1)

    gx = _emb_gather(emb, gx_pad)

    h1, av1 = _tc_pre(gx, W1, A1)
    u1, s1 = _edge_pass(h1, av1[:, 1], av1[:, 2], src, dst, zrows, zs)
    h2, av2 = _tc_mid(u1, s1.reshape(2, NPAD, 1), h1, av1,
                      b1.reshape(1, KGE), W2, A2)
    u2, s2 = _edge_pass(h2, av2[:, 1], av2[:, 2], src, dst, zrows, zs)
    hc = _tc_post(u2, s2.reshape(2, NPAD, 1), h2, av2,
                  b2.reshape(1, KGE), Wc, bc.reshape(1, KGE))

    return _tc_pool(hc, batch2d, hidden_states, seq_lengths,
                    Wd, bd.reshape(1, D), Wu, bu.reshape(1, H))


# speedup vs baseline: 33.2686x; 1.3044x over previous
import functools

import jax
import jax.numpy as jnp
from jax import lax
from jax.experimental import pallas as pl
from jax.experimental.pallas import tpu as pltpu
from jax.experimental.pallas import tpu_sc as plsc

N = 10000
E = 320000
B = 4
S = 2048
H = 1024
D = 128
KGE = 128
P = 32
NPAD = 10240
NTILES = 32
ROWS_PER_TILE = NPAD // NTILES
STRIPE = NPAD // 16
EPT = E // NTILES
EC = 80
NCHUNK = EPT // EC

_mesh = lambda: plsc.VectorSubcoreMesh(core_axis_name="c", subcore_axis_name="s")


def _emb_gather(emb, gx_pad):
    @functools.partial(
        pl.kernel,
        out_type=jax.ShapeDtypeStruct((NPAD, KGE), jnp.float32),
        mesh=_mesh(),
        scratch_types=[
            pltpu.VMEM((EC,), jnp.int32),
            pltpu.VMEM((EC, KGE), jnp.float32),
            pltpu.SemaphoreType.DMA,
        ],
    )
    def k(emb_hbm, idx_hbm, out_hbm, idx_v, rows_v, sem):
        cid = lax.axis_index("c")
        sid = lax.axis_index("s")
        tid = cid * 16 + sid
        base = tid * ROWS_PER_TILE
        for i in range(ROWS_PER_TILE // EC):
            off = pl.multiple_of(base + i * EC, 8)
            pltpu.sync_copy(idx_hbm.at[pl.ds(off, EC)], idx_v)
            pltpu.async_copy(emb_hbm.at[idx_v], rows_v, sem).wait()
            pltpu.sync_copy(rows_v, out_hbm.at[pl.ds(off, EC)])

    return k(emb, gx_pad)


def _edge_pass(h, asrc, adst, src, dst, zrows, zs):
    @functools.partial(
        pl.kernel,
        out_type=(
            jax.ShapeDtypeStruct((2, NPAD, KGE), jnp.float32),
            jax.ShapeDtypeStruct((2, NPAD), jnp.float32),
        ),
        mesh=_mesh(),
        scratch_types=[
            [pltpu.VMEM((EC,), jnp.int32)] * 2,
            [pltpu.VMEM((EC,), jnp.int32)] * 2,
            [pltpu.VMEM((EC,), jnp.float32)] * 2,
            [pltpu.VMEM((EC,), jnp.float32)] * 2,
            pltpu.VMEM((EC,), jnp.float32),
            [pltpu.VMEM((EC, KGE), jnp.float32)] * 2,
            pltpu.VMEM_SHARED((NPAD, KGE), jnp.float32),
            pltpu.VMEM_SHARED((NPAD,), jnp.float32),
            [pltpu.SemaphoreType.DMA] * 2,
            [pltpu.SemaphoreType.DMA] * 2,
            [pltpu.SemaphoreType.DMA] * 2,
        ],
    )
    def k(h_hbm, asrc_hbm, adst_hbm, src_hbm, dst_hbm, zr_hbm, zs_hbm,
          u_out, s_out, src_c, dst_c, a_v, b_v, ex_v, rows_v,
          u_sh, s_sh, sem_r, sem_s, sem_i):
        cid = lax.axis_index("c")
        sid = lax.axis_index("s")
        tid = cid * 16 + sid
        stripe = pl.multiple_of(sid * STRIPE, 8)
        ebase = tid * EPT
        pltpu.sync_copy(zr_hbm, u_sh.at[pl.ds(stripe, STRIPE)])
        pltpu.sync_copy(zs_hbm, s_sh.at[pl.ds(stripe, STRIPE)])
        plsc.subcore_barrier()

        def fire_idx(i, b, sync=False):
            off = pl.multiple_of(ebase + i * EC, 8)
            if sync:
                pltpu.sync_copy(src_hbm.at[pl.ds(off, EC)], src_c[b])
                pltpu.sync_copy(dst_hbm.at[pl.ds(off, EC)], dst_c[b])
            else:
                pltpu.async_copy(src_hbm.at[pl.ds(off, EC)], src_c[b],
                                 sem_i[b])
                pltpu.async_copy(dst_hbm.at[pl.ds(off, EC)], dst_c[b],
                                 sem_i[b])

        def fire_gather(b):
            pltpu.async_copy(h_hbm.at[src_c[b]], rows_v[b], sem_r[b])
            pltpu.async_copy(asrc_hbm.at[src_c[b]], a_v[b], sem_s[b])
            pltpu.async_copy(adst_hbm.at[dst_c[b]], b_v[b], sem_s[b])

        def wait_idx(b):
            pltpu.make_async_copy(src_hbm.at[pl.ds(0, EC)], src_c[b],
                                  sem_i[b]).wait()
            pltpu.make_async_copy(dst_hbm.at[pl.ds(0, EC)], dst_c[b],
                                  sem_i[b]).wait()

        def process(b):
            pltpu.make_async_copy(asrc_hbm.at[src_c[b]], a_v[b],
                                  sem_s[b]).wait()
            pltpu.make_async_copy(adst_hbm.at[dst_c[b]], b_v[b],
                                  sem_s[b]).wait()
            for g in range(EC // 16):
                x = a_v[b][pl.ds(g * 16, 16)] + b_v[b][pl.ds(g * 16, 16)]
                x = jnp.where(x > 0.0, x, 0.2 * x)
                ex_v[pl.ds(g * 16, 16)] = jnp.exp(x)
            pltpu.make_async_copy(h_hbm.at[src_c[b]], rows_v[b],
                                  sem_r[b]).wait()

            def rowgrp(g2, c2):
                exvec = ex_v[pl.ds(g2 * 16, 16)]
                for l in range(16):
                    w = jnp.full((16,), exvec[l], jnp.float32)
                    j = g2 * 16 + l
                    for k2 in range(KGE // 16):
                        rows_v[b][j, pl.ds(k2 * 16, 16)] = (
                            rows_v[b][j, pl.ds(k2 * 16, 16)] * w)
                return c2

            lax.fori_loop(0, EC // 16, rowgrp, 0)
            pltpu.sync_copy(rows_v[b], u_sh.at[dst_c[b]], add=True)
            pltpu.sync_copy(ex_v, s_sh.at[dst_c[b]], add=True)

        fire_idx(0, 0, sync=True)
        fire_gather(0)
        fire_idx(1, 1)

        def pair(i2, carry):
            for b in range(2):
                i = i2 * 2 + b
                b1 = 1 - b
                process(b)
                wait_idx(b1)
                fire_gather(b1)

                @pl.when(i + 2 < NCHUNK)
                def _():
                    fire_idx(i + 2, b)

            return carry

        lax.fori_loop(0, NCHUNK // 2, pair, 0)
        process(0)
        plsc.subcore_barrier()
        pltpu.sync_copy(u_sh.at[pl.ds(stripe, STRIPE)],
                        u_out.at[cid, pl.ds(stripe, STRIPE)])
        pltpu.sync_copy(s_sh.at[pl.ds(stripe, STRIPE)],
                        s_out.at[cid, pl.ds(stripe, STRIPE)])

    return k(h, asrc, adst, src, dst, zrows, zs)


def _tc_pre(x, W, A):
    def body(x_ref, w_ref, a_ref, h_ref, av_ref):
        hv = jnp.dot(x_ref[...], w_ref[...], preferred_element_type=jnp.float32)
        h_ref[...] = hv
        av_ref[...] = jnp.dot(hv, a_ref[...], preferred_element_type=jnp.float32)

    return pl.pallas_call(
        body,
        out_shape=(
            jax.ShapeDtypeStruct((NPAD, KGE), jnp.float32),
            jax.ShapeDtypeStruct((NPAD, KGE), jnp.float32),
        ),
    )(x, W, A)


def _tc_mid(u, s3, h, av, b_row, W2, A2):
    def body(u_ref, s_ref, h_ref, av_ref, b_ref, w_ref, a_ref, h2_ref, av2_ref):
        exl = jnp.exp(jnp.where(av_ref[:, 0:1] > 0.0, av_ref[:, 0:1],
                                0.2 * av_ref[:, 0:1]))
        stot = s_ref[0] + s_ref[1] + exl + 1e-16
        x2 = (u_ref[0] + u_ref[1] + exl * h_ref[...]) / stot + b_ref[...]
        x2 = jnp.maximum(x2, 0.0)
        h2 = jnp.dot(x2, w_ref[...], preferred_element_type=jnp.float32)
        h2_ref[...] = h2
        av2_ref[...] = jnp.dot(h2, a_ref[...], preferred_element_type=jnp.float32)

    return pl.pallas_call(
        body,
        out_shape=(
            jax.ShapeDtypeStruct((NPAD, KGE), jnp.float32),
            jax.ShapeDtypeStruct((NPAD, KGE), jnp.float32),
        ),
    )(u, s3, h, av, b_row, W2, A2)


def _tc_post(u, s3, h, av, b_row, Wc, bc_row):
    def body(u_ref, s_ref, h_ref, av_ref, b_ref, wc_ref, bc_ref, hc_ref):
        exl = jnp.exp(jnp.where(av_ref[:, 0:1] > 0.0, av_ref[:, 0:1],
                                0.2 * av_ref[:, 0:1]))
        stot = s_ref[0] + s_ref[1] + exl + 1e-16
        out2 = (u_ref[0] + u_ref[1] + exl * h_ref[...]) / stot + b_ref[...]
        hc_ref[...] = jnp.dot(out2, wc_ref[...],
                              preferred_element_type=jnp.float32) + bc_ref[...]

    return pl.pallas_call(
        body,
        out_shape=jax.ShapeDtypeStruct((NPAD, KGE), jnp.float32),
    )(u, s3, h, av, b_row, Wc, bc_row)


def _tc_pool(hc, batch2d, hidden, seq_lengths, Wd, bd_row, Wu, bu_row):
    def body(hc_ref, b2d_ref, hid_ref, sl_ref, wd_ref, bd_ref, wu_ref,
             bu_ref, out_ref):
        i = pl.program_id(0)
        b2d = b2d_ref[...]
        c = jnp.sum(jnp.where(b2d == i, 1, 0))
        off = jnp.sum(jnp.where(b2d < i, 1, 0))
        sl = sl_ref[i]
        L = c + sl
        p2 = lax.broadcasted_iota(jnp.int32, (P, 1), 0)
        s_ = (p2 * L) // P
        e_ = ((p2 + 1) * L + (P - 1)) // P
        lo_n = off + s_
        hi_n = off + jnp.minimum(e_, c)
        iota_n = lax.broadcasted_iota(jnp.int32, (P, NPAD), 1)
        mask_n = ((iota_n >= lo_n) & (iota_n < hi_n)).astype(jnp.float32)
        s_hc = jnp.dot(mask_n, hc_ref[...], preferred_element_type=jnp.float32)
        lo_h = jnp.maximum(s_, c) - c
        hi_h = jnp.maximum(e_ - c, 0)
        iota_s = lax.broadcasted_iota(jnp.int32, (P, S), 1)
        mask_h = ((iota_s >= lo_h) & (iota_s < hi_h)).astype(jnp.float32)
        s_raw = jnp.dot(mask_h, hid_ref[0], preferred_element_type=jnp.float32)
        s_hid = jnp.dot(s_raw, wd_ref[...], preferred_element_type=jnp.float32)
        nh = (hi_h - lo_h).astype(jnp.float32)
        w = (e_ - s_).astype(jnp.float32)
        pool = (s_hc + s_hid + nh * bd_ref[...]) / w
        silu = pool * (1.0 / (1.0 + jnp.exp(-pool)))
        out_ref[0] = jnp.dot(silu, wu_ref[...],
                             preferred_element_type=jnp.float32) + bu_ref[...]

    return pl.pallas_call(
        body,
        grid=(B,),
        in_specs=[
            pl.BlockSpec((NPAD, KGE), lambda i: (0, 0)),
            pl.BlockSpec((NPAD // 128, 128), lambda i: (0, 0)),
            pl.BlockSpec((1, S, H), lambda i: (i, 0, 0)),
            pl.BlockSpec(memory_space=pltpu.SMEM),
            pl.BlockSpec((H, D), lambda i: (0, 0)),
            pl.BlockSpec((1, D), lambda i: (0, 0)),
            pl.BlockSpec((D, H), lambda i: (0, 0)),
            pl.BlockSpec((1, H), lambda i: (0, 0)),
        ],
        out_specs=pl.BlockSpec((1, P, H), lambda i: (i, 0, 0)),
        out_shape=jax.ShapeDtypeStruct((B, P, H), jnp.float32),
    )(hc, batch2d, hidden, seq_lengths, Wd, bd_row, Wu, bu_row)


def kernel(graph_x, edge_index, batch, hidden_states, seq_lengths, emb,
           W1, as1, ad1, b1, W2, as2, ad2, b2, Wd, bd, Wc, bc, Wu, bu):
    f32 = jnp.float32
    gx_pad = jnp.pad(graph_x, (0, NPAD - N))
    src = edge_index[0]
    dst = edge_index[1]
    batch2d = jnp.pad(batch, (0, NPAD - N), constant_values=127).reshape(
        NPAD // 128, 128)
    zrows = jnp.zeros((STRIPE, KGE), f32)
    zs = jnp.zeros((STRIPE,), f32)
    zcol = jnp.zeros((KGE, KGE - 3), f32)
    A1 = jnp.concatenate(
        [(as1 + ad1)[:, None], as1[:, None], ad1[:, None], zcol], axis=1)
    A2 = jnp.concatenate(
        [(as2 + ad2)[:, None], as2[:, None], ad2[:, None], zcol], axis=1)

    gx = _emb_gather(emb, gx_pad)

    h1, av1 = _tc_pre(gx, W1, A1)
    u1, s1 = _edge_pass(h1, av1[:, 1], av1[:, 2], src, dst, zrows, zs)
    h2, av2 = _tc_mid(u1, s1.reshape(2, NPAD, 1), h1, av1,
                      b1.reshape(1, KGE), W2, A2)
    u2, s2 = _edge_pass(h2, av2[:, 1], av2[:, 2], src, dst, zrows, zs)
    hc = _tc_post(u2, s2.reshape(2, NPAD, 1), h2, av2,
                  b2.reshape(1, KGE), Wc, bc.reshape(1, KGE))

    return _tc_pool(hc, batch2d, hidden_states, seq_lengths,
                    Wd, bd.reshape(1, D), Wu, bu.reshape(1, H))


# speedup vs baseline: 40.4051x; 1.2145x over previous
import functools

import jax
import jax.numpy as jnp
from jax import lax
from jax.experimental import pallas as pl
from jax.experimental.pallas import tpu as pltpu
from jax.experimental.pallas import tpu_sc as plsc

N = 10000
E = 320000
B = 4
S = 2048
H = 1024
D = 128
KGE = 128
P = 32
NPAD = 10240
NTILES = 32
ROWS_PER_TILE = NPAD // NTILES
STRIPE = NPAD // 16
EPT = E // NTILES
EC = 80
NCHUNK = EPT // EC

_mesh = lambda: plsc.VectorSubcoreMesh(core_axis_name="c", subcore_axis_name="s")


def _emb_gather(emb, gx_pad):
    @functools.partial(
        pl.kernel,
        out_type=jax.ShapeDtypeStruct((NPAD, KGE), jnp.float32),
        mesh=_mesh(),
        scratch_types=[
            pltpu.VMEM((EC,), jnp.int32),
            pltpu.VMEM((EC, KGE), jnp.float32),
            pltpu.SemaphoreType.DMA,
        ],
    )
    def k(emb_hbm, idx_hbm, out_hbm, idx_v, rows_v, sem):
        cid = lax.axis_index("c")
        sid = lax.axis_index("s")
        tid = cid * 16 + sid
        base = tid * ROWS_PER_TILE
        for i in range(ROWS_PER_TILE // EC):
            off = pl.multiple_of(base + i * EC, 8)
            pltpu.sync_copy(idx_hbm.at[pl.ds(off, EC)], idx_v)
            pltpu.async_copy(emb_hbm.at[idx_v], rows_v, sem).wait()
            pltpu.sync_copy(rows_v, out_hbm.at[pl.ds(off, EC)])

    return k(emb, gx_pad)


def _edge_pass(h, asrc, adst, src, dst, zrows, zs):
    @functools.partial(
        pl.kernel,
        out_type=(
            jax.ShapeDtypeStruct((2, NPAD, KGE), jnp.float32),
            jax.ShapeDtypeStruct((2, NPAD), jnp.float32),
        ),
        mesh=_mesh(),
        scratch_types=[
            [pltpu.VMEM((EC,), jnp.int32)] * 3,
            [pltpu.VMEM((EC,), jnp.int32)] * 3,
            [pltpu.VMEM((EC,), jnp.float32)] * 3,
            [pltpu.VMEM((EC,), jnp.float32)] * 3,
            [pltpu.VMEM((EC,), jnp.float32)] * 3,
            [pltpu.VMEM((EC, KGE), jnp.float32)] * 3,
            pltpu.VMEM_SHARED((NPAD, KGE), jnp.float32),
            pltpu.VMEM_SHARED((NPAD,), jnp.float32),
            [pltpu.SemaphoreType.DMA] * 3,
            [pltpu.SemaphoreType.DMA] * 3,
            [pltpu.SemaphoreType.DMA] * 3,
            [pltpu.SemaphoreType.DMA] * 3,
        ],
    )
    def k(h_hbm, asrc_hbm, adst_hbm, src_hbm, dst_hbm, zr_hbm, zs_hbm,
          u_out, s_out, src_c, dst_c, a_v, b_v, ex_v, rows_v,
          u_sh, s_sh, sem_r, sem_s, sem_i, sem_w):
        cid = lax.axis_index("c")
        sid = lax.axis_index("s")
        tid = cid * 16 + sid
        stripe = pl.multiple_of(sid * STRIPE, 8)
        ebase = tid * EPT
        pltpu.sync_copy(zr_hbm, u_sh.at[pl.ds(stripe, STRIPE)])
        pltpu.sync_copy(zs_hbm, s_sh.at[pl.ds(stripe, STRIPE)])
        plsc.subcore_barrier()

        def fire_idx(i, b, sync=False):
            off = pl.multiple_of(ebase + i * EC, 8)
            if sync:
                pltpu.sync_copy(src_hbm.at[pl.ds(off, EC)], src_c[b])
                pltpu.sync_copy(dst_hbm.at[pl.ds(off, EC)], dst_c[b])
            else:
                pltpu.async_copy(src_hbm.at[pl.ds(off, EC)], src_c[b],
                                 sem_i[b])
                pltpu.async_copy(dst_hbm.at[pl.ds(off, EC)], dst_c[b],
                                 sem_i[b])

        def fire_gather(b):
            pltpu.async_copy(h_hbm.at[src_c[b]], rows_v[b], sem_r[b])
            pltpu.async_copy(asrc_hbm.at[src_c[b]], a_v[b], sem_s[b])
            pltpu.async_copy(adst_hbm.at[dst_c[b]], b_v[b], sem_s[b])

        def wait_idx(b):
            pltpu.make_async_copy(src_hbm.at[pl.ds(0, EC)], src_c[b],
                                  sem_i[b]).wait()
            pltpu.make_async_copy(dst_hbm.at[pl.ds(0, EC)], dst_c[b],
                                  sem_i[b]).wait()

        def wait_scat(b):
            pltpu.make_async_copy(rows_v[b], u_sh.at[dst_c[b]],
                                  sem_w[b]).wait()
            pltpu.make_async_copy(ex_v[b], s_sh.at[dst_c[b]],
                                  sem_w[b]).wait()

        def process(b):
            pltpu.make_async_copy(asrc_hbm.at[src_c[b]], a_v[b],
                                  sem_s[b]).wait()
            pltpu.make_async_copy(adst_hbm.at[dst_c[b]], b_v[b],
                                  sem_s[b]).wait()
            for g in range(EC // 16):
                x = a_v[b][pl.ds(g * 16, 16)] + b_v[b][pl.ds(g * 16, 16)]
                x = jnp.where(x > 0.0, x, 0.2 * x)
                ex_v[b][pl.ds(g * 16, 16)] = jnp.exp(x)
            pltpu.make_async_copy(h_hbm.at[src_c[b]], rows_v[b],
                                  sem_r[b]).wait()

            def rowgrp(g2, c2):
                exvec = ex_v[b][pl.ds(g2 * 16, 16)]
                for l in range(16):
                    w = jnp.full((16,), exvec[l], jnp.float32)
                    j = g2 * 16 + l
                    for k2 in range(KGE // 16):
                        rows_v[b][j, pl.ds(k2 * 16, 16)] = (
                            rows_v[b][j, pl.ds(k2 * 16, 16)] * w)
                return c2

            lax.fori_loop(0, EC // 16, rowgrp, 0)
            pltpu.async_copy(rows_v[b], u_sh.at[dst_c[b]], sem_w[b],
                             add=True)
            pltpu.async_copy(ex_v[b], s_sh.at[dst_c[b]], sem_w[b],
                             add=True)

        def body(i, p, p1, p2):
            process(p)

            @pl.when(i + 2 < NCHUNK)
            def _():
                @pl.when(i >= 1)
                def _():
                    wait_scat(p2)

                fire_idx(i + 2, p2)

            @pl.when(i + 1 < NCHUNK)
            def _():
                wait_idx(p1)
                fire_gather(p1)

        fire_idx(0, 0, sync=True)
        fire_gather(0)
        fire_idx(1, 1)

        def triple(i3, carry):
            for b in range(3):
                body(i3 * 3 + b, b, (b + 1) % 3, (b + 2) % 3)
            return carry

        lax.fori_loop(0, NCHUNK // 3, triple, 0)
        body(NCHUNK - 2, 0, 1, 2)
        process(1)
        wait_scat(2)
        wait_scat(0)
        wait_scat(1)
        plsc.subcore_barrier()
        pltpu.sync_copy(u_sh.at[pl.ds(stripe, STRIPE)],
                        u_out.at[cid, pl.ds(stripe, STRIPE)])
        pltpu.sync_copy(s_sh.at[pl.ds(stripe, STRIPE)],
                        s_out.at[cid, pl.ds(stripe, STRIPE)])

    return k(h, asrc, adst, src, dst, zrows, zs)


def _tc_pre(x, W, A):
    def body(x_ref, w_ref, a_ref, h_ref, av_ref):
        hv = jnp.dot(x_ref[...], w_ref[...], preferred_element_type=jnp.float32)
        h_ref[...] = hv
        av_ref[...] = jnp.dot(hv, a_ref[...], preferred_element_type=jnp.float32)

    return pl.pallas_call(
        body,
        out_shape=(
            jax.ShapeDtypeStruct((NPAD, KGE), jnp.float32),
            jax.ShapeDtypeStruct((NPAD, KGE), jnp.float32),
        ),
    )(x, W, A)


def _tc_mid(u, s3, h, av, b_row, W2, A2):
    def body(u_ref, s_ref, h_ref, av_ref, b_ref, w_ref, a_ref, h2_ref, av2_ref):
        exl = jnp.exp(jnp.where(av_ref[:, 0:1] > 0.0, av_ref[:, 0:1],
                                0.2 * av_ref[:, 0:1]))
        stot = s_ref[0] + s_ref[1] + exl + 1e-16
        x2 = (u_ref[0] + u_ref[1] + exl * h_ref[...]) / stot + b_ref[...]
        x2 = jnp.maximum(x2, 0.0)
        h2 = jnp.dot(x2, w_ref[...], preferred_element_type=jnp.float32)
        h2_ref[...] = h2
        av2_ref[...] = jnp.dot(h2, a_ref[...], preferred_element_type=jnp.float32)

    return pl.pallas_call(
        body,
        out_shape=(
            jax.ShapeDtypeStruct((NPAD, KGE), jnp.float32),
            jax.ShapeDtypeStruct((NPAD, KGE), jnp.float32),
        ),
    )(u, s3, h, av, b_row, W2, A2)


def _tc_post(u, s3, h, av, b_row, Wc, bc_row):
    def body(u_ref, s_ref, h_ref, av_ref, b_ref, wc_ref, bc_ref, hc_ref):
        exl = jnp.exp(jnp.where(av_ref[:, 0:1] > 0.0, av_ref[:, 0:1],
                                0.2 * av_ref[:, 0:1]))
        stot = s_ref[0] + s_ref[1] + exl + 1e-16
        out2 = (u_ref[0] + u_ref[1] + exl * h_ref[...]) / stot + b_ref[...]
        hc_ref[...] = jnp.dot(out2, wc_ref[...],
                              preferred_element_type=jnp.float32) + bc_ref[...]

    return pl.pallas_call(
        body,
        out_shape=jax.ShapeDtypeStruct((NPAD, KGE), jnp.float32),
    )(u, s3, h, av, b_row, Wc, bc_row)


def _tc_pool(hc, batch2d, hidden, seq_lengths, Wd, bd_row, Wu, bu_row):
    def body(hc_ref, b2d_ref, hid_ref, sl_ref, wd_ref, bd_ref, wu_ref,
             bu_ref, out_ref):
        i = pl.program_id(0)
        b2d = b2d_ref[...]
        c = jnp.sum(jnp.where(b2d == i, 1, 0))
        off = jnp.sum(jnp.where(b2d < i, 1, 0))
        sl = sl_ref[i]
        L = c + sl
        p2 = lax.broadcasted_iota(jnp.int32, (P, 1), 0)
        s_ = (p2 * L) // P
        e_ = ((p2 + 1) * L + (P - 1)) // P
        lo_n = off + s_
        hi_n = off + jnp.minimum(e_, c)
        iota_n = lax.broadcasted_iota(jnp.int32, (P, NPAD), 1)
        mask_n = ((iota_n >= lo_n) & (iota_n < hi_n)).astype(jnp.float32)
        s_hc = jnp.dot(mask_n, hc_ref[...], preferred_element_type=jnp.float32)
        lo_h = jnp.maximum(s_, c) - c
        hi_h = jnp.maximum(e_ - c, 0)
        iota_s = lax.broadcasted_iota(jnp.int32, (P, S), 1)
        mask_h = ((iota_s >= lo_h) & (iota_s < hi_h)).astype(jnp.float32)
        s_raw = jnp.dot(mask_h, hid_ref[0], preferred_element_type=jnp.float32)
        s_hid = jnp.dot(s_raw, wd_ref[...], preferred_element_type=jnp.float32)
        nh = (hi_h - lo_h).astype(jnp.float32)
        w = (e_ - s_).astype(jnp.float32)
        pool = (s_hc + s_hid + nh * bd_ref[...]) / w
        silu = pool * (1.0 / (1.0 + jnp.exp(-pool)))
        out_ref[0] = jnp.dot(silu, wu_ref[...],
                             preferred_element_type=jnp.float32) + bu_ref[...]

    return pl.pallas_call(
        body,
        grid=(B,),
        in_specs=[
            pl.BlockSpec((NPAD, KGE), lambda i: (0, 0)),
            pl.BlockSpec((NPAD // 128, 128), lambda i: (0, 0)),
            pl.BlockSpec((1, S, H), lambda i: (i, 0, 0)),
            pl.BlockSpec(memory_space=pltpu.SMEM),
            pl.BlockSpec((H, D), lambda i: (0, 0)),
            pl.BlockSpec((1, D), lambda i: (0, 0)),
            pl.BlockSpec((D, H), lambda i: (0, 0)),
            pl.BlockSpec((1, H), lambda i: (0, 0)),
        ],
        out_specs=pl.BlockSpec((1, P, H), lambda i: (i, 0, 0)),
        out_shape=jax.ShapeDtypeStruct((B, P, H), jnp.float32),
    )(hc, batch2d, hidden, seq_lengths, Wd, bd_row, Wu, bu_row)


def kernel(graph_x, edge_index, batch, hidden_states, seq_lengths, emb,
           W1, as1, ad1, b1, W2, as2, ad2, b2, Wd, bd, Wc, bc, Wu, bu):
    f32 = jnp.float32
    gx_pad = jnp.pad(graph_x, (0, NPAD - N))
    src = edge_index[0]
    dst = edge_index[1]
    batch2d = jnp.pad(batch, (0, NPAD - N), constant_values=127).reshape(
        NPAD // 128, 128)
    zrows = jnp.zeros((STRIPE, KGE), f32)
    zs = jnp.zeros((STRIPE,), f32)
    zcol = jnp.zeros((KGE, KGE - 3), f32)
    A1 = jnp.concatenate(
        [(as1 + ad1)[:, None], as1[:, None], ad1[:, None], zcol], axis=1)
    A2 = jnp.concatenate(
        [(as2 + ad2)[:, None], as2[:, None], ad2[:, None], zcol], axis=1)

    gx = _emb_gather(emb, gx_pad)

    h1, av1 = _tc_pre(gx, W1, A1)
    u1, s1 = _edge_pass(h1, av1[:, 1], av1[:, 2], src, dst, zrows, zs)
    h2, av2 = _tc_mid(u1, s1.reshape(2, NPAD, 1), h1, av1,
                      b1.reshape(1, KGE), W2, A2)
    u2, s2 = _edge_pass(h2, av2[:, 1], av2[:, 2], src, dst, zrows, zs)
    hc = _tc_post(u2, s2.reshape(2, NPAD, 1), h2, av2,
                  b2.reshape(1, KGE), Wc, bc.reshape(1, KGE))

    return _tc_pool(hc, batch2d, hidden_states, seq_lengths,
                    Wd, bd.reshape(1, D), Wu, bu.reshape(1, H))


# speedup vs baseline: 57.1427x; 1.4142x over previous
import functools

import jax
import jax.numpy as jnp
from jax import lax
from jax.experimental import pallas as pl
from jax.experimental.pallas import tpu as pltpu
from jax.experimental.pallas import tpu_sc as plsc

N = 10000
E = 320000
B = 4
S = 2048
H = 1024
D = 128
KGE = 128
P = 32
NPAD = 10240
NTILES = 32
ROWS_PER_TILE = NPAD // NTILES
STRIPE = NPAD // 16
EPT = E // NTILES
EC = 80
NCHUNK = EPT // EC

_mesh = lambda: plsc.VectorSubcoreMesh(core_axis_name="c", subcore_axis_name="s")


def _emb_gather(emb, gx_pad):
    @functools.partial(
        pl.kernel,
        out_type=jax.ShapeDtypeStruct((NPAD, KGE), jnp.float32),
        mesh=_mesh(),
        scratch_types=[
            pltpu.VMEM((EC,), jnp.int32),
            pltpu.VMEM((EC, KGE), jnp.float32),
            pltpu.SemaphoreType.DMA,
        ],
    )
    def k(emb_hbm, idx_hbm, out_hbm, idx_v, rows_v, sem):
        cid = lax.axis_index("c")
        sid = lax.axis_index("s")
        tid = cid * 16 + sid
        base = tid * ROWS_PER_TILE
        for i in range(ROWS_PER_TILE // EC):
            off = pl.multiple_of(base + i * EC, 8)
            pltpu.sync_copy(idx_hbm.at[pl.ds(off, EC)], idx_v)
            pltpu.async_copy(emb_hbm.at[idx_v], rows_v, sem).wait()
            pltpu.sync_copy(rows_v, out_hbm.at[pl.ds(off, EC)])

    return k(emb, gx_pad)


def _edge_pass(h, asrc, adst, src, dst, zrows, zs):
    @functools.partial(
        pl.kernel,
        out_type=(
            jax.ShapeDtypeStruct((2, NPAD, KGE), jnp.float32),
            jax.ShapeDtypeStruct((2, NPAD), jnp.float32),
        ),
        mesh=_mesh(),
        scratch_types=[
            [pltpu.VMEM((EC,), jnp.int32)] * 3,
            [pltpu.VMEM((EC,), jnp.int32)] * 3,
            [pltpu.VMEM((EC,), jnp.int32)] * 3,
            [pltpu.VMEM((EC,), jnp.float32)] * 3,
            [pltpu.VMEM((EC,), jnp.float32)] * 3,
            [pltpu.VMEM((EC,), jnp.float32)] * 3,
            [pltpu.VMEM((EC, KGE), jnp.float32)] * 3,
            pltpu.VMEM_SHARED((NPAD, KGE), jnp.float32),
            pltpu.VMEM_SHARED((NPAD,), jnp.float32),
            [pltpu.SemaphoreType.DMA] * 3,
            [pltpu.SemaphoreType.DMA] * 3,
            [pltpu.SemaphoreType.DMA] * 3,
            [pltpu.SemaphoreType.DMA] * 3,
        ],
    )
    def k(h_hbm, asrc_hbm, adst_hbm, src_hbm, dst_hbm, zr_hbm, zs_hbm,
          u_out, s_out, src_c, dst_c, dst_s, a_v, b_v, ex_v, rows_v,
          u_sh, s_sh, sem_r, sem_s, sem_i, sem_w):
        cid = lax.axis_index("c")
        sid = lax.axis_index("s")
        tid = cid * 16 + sid
        stripe = pl.multiple_of(sid * STRIPE, 8)
        ebase = tid * EPT
        pltpu.sync_copy(zr_hbm, u_sh.at[pl.ds(stripe, STRIPE)])
        pltpu.sync_copy(zs_hbm, s_sh.at[pl.ds(stripe, STRIPE)])
        plsc.subcore_barrier()

        def fire_idx(i, b, sync=False):
            off = pl.multiple_of(ebase + i * EC, 8)
            if sync:
                pltpu.sync_copy(src_hbm.at[pl.ds(off, EC)], src_c[b])
                pltpu.sync_copy(dst_hbm.at[pl.ds(off, EC)], dst_c[b])
            else:
                pltpu.async_copy(src_hbm.at[pl.ds(off, EC)], src_c[b],
                                 sem_i[b])
                pltpu.async_copy(dst_hbm.at[pl.ds(off, EC)], dst_c[b],
                                 sem_i[b])

        def fire_gather(b):
            pltpu.async_copy(h_hbm.at[src_c[b]], rows_v[b], sem_r[b])
            pltpu.async_copy(asrc_hbm.at[src_c[b]], a_v[b], sem_s[b])
            pltpu.async_copy(adst_hbm.at[dst_c[b]], b_v[b], sem_s[b])

        def wait_idx(b):
            pltpu.make_async_copy(src_hbm.at[pl.ds(0, EC)], src_c[b],
                                  sem_i[b]).wait()
            pltpu.make_async_copy(dst_hbm.at[pl.ds(0, EC)], dst_c[b],
                                  sem_i[b]).wait()

        def wait_scat(b):
            pltpu.make_async_copy(rows_v[b], u_sh.at[dst_s[b]],
                                  sem_w[b]).wait()
            pltpu.make_async_copy(ex_v[b], s_sh.at[dst_s[b]],
                                  sem_w[b]).wait()

        def process(b):
            pltpu.make_async_copy(asrc_hbm.at[src_c[b]], a_v[b],
                                  sem_s[b]).wait()
            pltpu.make_async_copy(adst_hbm.at[dst_c[b]], b_v[b],
                                  sem_s[b]).wait()
            for g in range(EC // 16):
                x = a_v[b][pl.ds(g * 16, 16)] + b_v[b][pl.ds(g * 16, 16)]
                x = jnp.where(x > 0.0, x, 0.2 * x)
                ex_v[b][pl.ds(g * 16, 16)] = jnp.exp(x)
            pltpu.make_async_copy(h_hbm.at[src_c[b]], rows_v[b],
                                  sem_r[b]).wait()

            def rowgrp(g2, c2):
                exvec = ex_v[b][pl.ds(g2 * 16, 16)]
                for l in range(16):
                    w = jnp.full((16,), exvec[l], jnp.float32)
                    j = g2 * 16 + l
                    for k2 in range(KGE // 16):
                        rows_v[b][j, pl.ds(k2 * 16, 16)] = (
                            rows_v[b][j, pl.ds(k2 * 16, 16)] * w)
                return c2

            lax.fori_loop(0, EC // 16, rowgrp, 0)
            for g in range(EC // 16):
                dst_s[b][pl.ds(g * 16, 16)] = dst_c[b][pl.ds(g * 16, 16)]
            pltpu.async_copy(rows_v[b], u_sh.at[dst_s[b]], sem_w[b],
                             add=True)
            pltpu.async_copy(ex_v[b], s_sh.at[dst_s[b]], sem_w[b],
                             add=True)

        def body(i, p, p1, p2):
            @pl.when(i + 2 < NCHUNK)
            def _():
                fire_idx(i + 2, p2)

            @pl.when(i + 1 < NCHUNK)
            def _():
                @pl.when(i >= 2)
                def _():
                    wait_scat(p1)

                wait_idx(p1)
                fire_gather(p1)

            process(p)

        fire_idx(0, 0, sync=True)
        fire_gather(0)
        fire_idx(1, 1)

        def triple(i3, carry):
            for b in range(3):
                body(i3 * 3 + b, b, (b + 1) % 3, (b + 2) % 3)
            return carry

        lax.fori_loop(0, NCHUNK // 3, triple, 0)
        body(NCHUNK - 2, 0, 1, 2)
        process(1)
        wait_scat(2)
        wait_scat(0)
        wait_scat(1)
        plsc.subcore_barrier()
        pltpu.sync_copy(u_sh.at[pl.ds(stripe, STRIPE)],
                        u_out.at[cid, pl.ds(stripe, STRIPE)])
        pltpu.sync_copy(s_sh.at[pl.ds(stripe, STRIPE)],
                        s_out.at[cid, pl.ds(stripe, STRIPE)])

    return k(h, asrc, adst, src, dst, zrows, zs)


def _tc_pre(x, W, A):
    def body(x_ref, w_ref, a_ref, h_ref, av_ref):
        hv = jnp.dot(x_ref[...], w_ref[...], preferred_element_type=jnp.float32)
        h_ref[...] = hv
        av_ref[...] = jnp.dot(hv, a_ref[...], preferred_element_type=jnp.float32)

    return pl.pallas_call(
        body,
        out_shape=(
            jax.ShapeDtypeStruct((NPAD, KGE), jnp.float32),
            jax.ShapeDtypeStruct((NPAD, KGE), jnp.float32),
        ),
    )(x, W, A)


def _tc_mid(u, s3, h, av, b_row, W2, A2):
    def body(u_ref, s_ref, h_ref, av_ref, b_ref, w_ref, a_ref, h2_ref, av2_ref):
        exl = jnp.exp(jnp.where(av_ref[:, 0:1] > 0.0, av_ref[:, 0:1],
                                0.2 * av_ref[:, 0:1]))
        stot = s_ref[0] + s_ref[1] + exl + 1e-16
        x2 = (u_ref[0] + u_ref[1] + exl * h_ref[...]) / stot + b_ref[...]
        x2 = jnp.maximum(x2, 0.0)
        h2 = jnp.dot(x2, w_ref[...], preferred_element_type=jnp.float32)
        h2_ref[...] = h2
        av2_ref[...] = jnp.dot(h2, a_ref[...], preferred_element_type=jnp.float32)

    return pl.pallas_call(
        body,
        out_shape=(
            jax.ShapeDtypeStruct((NPAD, KGE), jnp.float32),
            jax.ShapeDtypeStruct((NPAD, KGE), jnp.float32),
        ),
    )(u, s3, h, av, b_row, W2, A2)


def _tc_post(u, s3, h, av, b_row, Wc, bc_row):
    def body(u_ref, s_ref, h_ref, av_ref, b_ref, wc_ref, bc_ref, hc_ref):
        exl = jnp.exp(jnp.where(av_ref[:, 0:1] > 0.0, av_ref[:, 0:1],
                                0.2 * av_ref[:, 0:1]))
        stot = s_ref[0] + s_ref[1] + exl + 1e-16
        out2 = (u_ref[0] + u_ref[1] + exl * h_ref[...]) / stot + b_ref[...]
        hc_ref[...] = jnp.dot(out2, wc_ref[...],
                              preferred_element_type=jnp.float32) + bc_ref[...]

    return pl.pallas_call(
        body,
        out_shape=jax.ShapeDtypeStruct((NPAD, KGE), jnp.float32),
    )(u, s3, h, av, b_row, Wc, bc_row)


def _tc_pool(hc, batch2d, hidden, seq_lengths, Wd, bd_row, Wu, bu_row):
    def body(hc_ref, b2d_ref, hid_ref, sl_ref, wd_ref, bd_ref, wu_ref,
             bu_ref, out_ref):
        i = pl.program_id(0)
        b2d = b2d_ref[...]
        c = jnp.sum(jnp.where(b2d == i, 1, 0))
        off = jnp.sum(jnp.where(b2d < i, 1, 0))
        sl = sl_ref[i]
        L = c + sl
        p2 = lax.broadcasted_iota(jnp.int32, (P, 1), 0)
        s_ = (p2 * L) // P
        e_ = ((p2 + 1) * L + (P - 1)) // P
        lo_n = off + s_
        hi_n = off + jnp.minimum(e_, c)
        iota_n = lax.broadcasted_iota(jnp.int32, (P, NPAD), 1)
        mask_n = ((iota_n >= lo_n) & (iota_n < hi_n)).astype(jnp.float32)
        s_hc = jnp.dot(mask_n, hc_ref[...], preferred_element_type=jnp.float32)
        lo_h = jnp.maximum(s_, c) - c
        hi_h = jnp.maximum(e_ - c, 0)
        iota_s = lax.broadcasted_iota(jnp.int32, (P, S), 1)
        mask_h = ((iota_s >= lo_h) & (iota_s < hi_h)).astype(jnp.float32)
        s_raw = jnp.dot(mask_h, hid_ref[0], preferred_element_type=jnp.float32)
        s_hid = jnp.dot(s_raw, wd_ref[...], preferred_element_type=jnp.float32)
        nh = (hi_h - lo_h).astype(jnp.float32)
        w = (e_ - s_).astype(jnp.float32)
        pool = (s_hc + s_hid + nh * bd_ref[...]) / w
        silu = pool * (1.0 / (1.0 + jnp.exp(-pool)))
        out_ref[0] = jnp.dot(silu, wu_ref[...],
                             preferred_element_type=jnp.float32) + bu_ref[...]

    return pl.pallas_call(
        body,
        grid=(B,),
        in_specs=[
            pl.BlockSpec((NPAD, KGE), lambda i: (0, 0)),
            pl.BlockSpec((NPAD // 128, 128), lambda i: (0, 0)),
            pl.BlockSpec((1, S, H), lambda i: (i, 0, 0)),
            pl.BlockSpec(memory_space=pltpu.SMEM),
            pl.BlockSpec((H, D), lambda i: (0, 0)),
            pl.BlockSpec((1, D), lambda i: (0, 0)),
            pl.BlockSpec((D, H), lambda i: (0, 0)),
            pl.BlockSpec((1, H), lambda i: (0, 0)),
        ],
        out_specs=pl.BlockSpec((1, P, H), lambda i: (i, 0, 0)),
        out_shape=jax.ShapeDtypeStruct((B, P, H), jnp.float32),
    )(hc, batch2d, hidden, seq_lengths, Wd, bd_row, Wu, bu_row)


def kernel(graph_x, edge_index, batch, hidden_states, seq_lengths, emb,
           W1, as1, ad1, b1, W2, as2, ad2, b2, Wd, bd, Wc, bc, Wu, bu):
    f32 = jnp.float32
    gx_pad = jnp.pad(graph_x, (0, NPAD - N))
    src = edge_index[0]
    dst = edge_index[1]
    batch2d = jnp.pad(batch, (0, NPAD - N), constant_values=127).reshape(
        NPAD // 128, 128)
    zrows = jnp.zeros((STRIPE, KGE), f32)
    zs = jnp.zeros((STRIPE,), f32)
    zcol = jnp.zeros((KGE, KGE - 3), f32)
    A1 = jnp.concatenate(
        [(as1 + ad1)[:, None], as1[:, None], ad1[:, None], zcol], axis=1)
    A2 = jnp.concatenate(
        [(as2 + ad2)[:, None], as2[:, None], ad2[:, None], zcol], axis=1)

    gx = _emb_gather(emb, gx_pad)

    h1, av1 = _tc_pre(gx, W1, A1)
    u1, s1 = _edge_pass(h1, av1[:, 1], av1[:, 2], src, dst, zrows, zs)
    h2, av2 = _tc_mid(u1, s1.reshape(2, NPAD, 1), h1, av1,
                      b1.reshape(1, KGE), W2, A2)
    u2, s2 = _edge_pass(h2, av2[:, 1], av2[:, 2], src, dst, zrows, zs)
    hc = _tc_post(u2, s2.reshape(2, NPAD, 1), h2, av2,
                  b2.reshape(1, KGE), Wc, bc.reshape(1, KGE))

    return _tc_pool(hc, batch2d, hidden_states, seq_lengths,
                    Wd, bd.reshape(1, D), Wu, bu.reshape(1, H))


# speedup vs baseline: 58.6915x; 1.0271x over previous
import functools

import jax
import jax.numpy as jnp
from jax import lax
from jax.experimental import pallas as pl
from jax.experimental.pallas import tpu as pltpu
from jax.experimental.pallas import tpu_sc as plsc

N = 10000
E = 320000
B = 4
S = 2048
H = 1024
D = 128
KGE = 128
P = 32
NPAD = 10240
NTILES = 32
ROWS_PER_TILE = NPAD // NTILES
STRIPE = NPAD // 16
EPT = E // NTILES
EC = 80
NCHUNK = EPT // EC

_mesh = lambda: plsc.VectorSubcoreMesh(core_axis_name="c", subcore_axis_name="s")


def _emb_gather(emb, gx_pad):
    @functools.partial(
        pl.kernel,
        out_type=jax.ShapeDtypeStruct((NPAD, KGE), jnp.float32),
        mesh=_mesh(),
        scratch_types=[
            pltpu.VMEM((EC,), jnp.int32),
            pltpu.VMEM((EC, KGE), jnp.float32),
            pltpu.SemaphoreType.DMA,
        ],
    )
    def k(emb_hbm, idx_hbm, out_hbm, idx_v, rows_v, sem):
        cid = lax.axis_index("c")
        sid = lax.axis_index("s")
        tid = cid * 16 + sid
        base = tid * ROWS_PER_TILE
        for i in range(ROWS_PER_TILE // EC):
            off = pl.multiple_of(base + i * EC, 8)
            pltpu.sync_copy(idx_hbm.at[pl.ds(off, EC)], idx_v)
            pltpu.async_copy(emb_hbm.at[idx_v], rows_v, sem).wait()
            pltpu.sync_copy(rows_v, out_hbm.at[pl.ds(off, EC)])

    return k(emb, gx_pad)


def _edge_pass(h, asrc, adst, src, dst, zrows, zs):
    @functools.partial(
        pl.kernel,
        out_type=(
            jax.ShapeDtypeStruct((2, NPAD, KGE), jnp.float32),
            jax.ShapeDtypeStruct((2, NPAD), jnp.float32),
        ),
        mesh=_mesh(),
        scratch_types=[
            [pltpu.VMEM((EC,), jnp.int32)] * 4,
            [pltpu.VMEM((EC,), jnp.int32)] * 4,
            [pltpu.VMEM((EC,), jnp.int32)] * 4,
            [pltpu.VMEM((EC,), jnp.float32)] * 4,
            [pltpu.VMEM((EC,), jnp.float32)] * 4,
            [pltpu.VMEM((EC,), jnp.float32)] * 4,
            [pltpu.VMEM((EC, KGE), jnp.float32)] * 4,
            pltpu.VMEM_SHARED((NPAD, KGE), jnp.float32),
            pltpu.VMEM_SHARED((NPAD,), jnp.float32),
            [pltpu.SemaphoreType.DMA] * 4,
            [pltpu.SemaphoreType.DMA] * 4,
            [pltpu.SemaphoreType.DMA] * 4,
            [pltpu.SemaphoreType.DMA] * 4,
        ],
    )
    def k(h_hbm, asrc_hbm, adst_hbm, src_hbm, dst_hbm, zr_hbm, zs_hbm,
          u_out, s_out, src_c, dst_c, dst_s, a_v, b_v, ex_v, rows_v,
          u_sh, s_sh, sem_r, sem_s, sem_i, sem_w):
        cid = lax.axis_index("c")
        sid = lax.axis_index("s")
        tid = cid * 16 + sid
        stripe = pl.multiple_of(sid * STRIPE, 8)
        ebase = tid * EPT
        pltpu.sync_copy(zr_hbm, u_sh.at[pl.ds(stripe, STRIPE)])
        pltpu.sync_copy(zs_hbm, s_sh.at[pl.ds(stripe, STRIPE)])
        plsc.subcore_barrier()

        def fire_idx(i, b, sync=False):
            off = pl.multiple_of(ebase + i * EC, 8)
            if sync:
                pltpu.sync_copy(src_hbm.at[pl.ds(off, EC)], src_c[b])
                pltpu.sync_copy(dst_hbm.at[pl.ds(off, EC)], dst_c[b])
            else:
                pltpu.async_copy(src_hbm.at[pl.ds(off, EC)], src_c[b],
                                 sem_i[b])
                pltpu.async_copy(dst_hbm.at[pl.ds(off, EC)], dst_c[b],
                                 sem_i[b])

        def fire_gather(b):
            pltpu.async_copy(h_hbm.at[src_c[b]], rows_v[b], sem_r[b])
            pltpu.async_copy(asrc_hbm.at[src_c[b]], a_v[b], sem_s[b])
            pltpu.async_copy(adst_hbm.at[dst_c[b]], b_v[b], sem_s[b])

        def wait_idx(b):
            pltpu.make_async_copy(src_hbm.at[pl.ds(0, EC)], src_c[b],
                                  sem_i[b]).wait()
            pltpu.make_async_copy(dst_hbm.at[pl.ds(0, EC)], dst_c[b],
                                  sem_i[b]).wait()

        def wait_scat(b):
            pltpu.make_async_copy(rows_v[b], u_sh.at[dst_s[b]],
                                  sem_w[b]).wait()
            pltpu.make_async_copy(ex_v[b], s_sh.at[dst_s[b]],
                                  sem_w[b]).wait()

        def process(b):
            pltpu.make_async_copy(asrc_hbm.at[src_c[b]], a_v[b],
                                  sem_s[b]).wait()
            pltpu.make_async_copy(adst_hbm.at[dst_c[b]], b_v[b],
                                  sem_s[b]).wait()
            for g in range(EC // 16):
                x = a_v[b][pl.ds(g * 16, 16)] + b_v[b][pl.ds(g * 16, 16)]
                x = jnp.where(x > 0.0, x, 0.2 * x)
                ex_v[b][pl.ds(g * 16, 16)] = jnp.exp(x)
            pltpu.make_async_copy(h_hbm.at[src_c[b]], rows_v[b],
                                  sem_r[b]).wait()

            def rowgrp(g2, c2):
                exvec = ex_v[b][pl.ds(g2 * 16, 16)]
                for l in range(16):
                    w = jnp.full((16,), exvec[l], jnp.float32)
                    j = g2 * 16 + l
                    for k2 in range(KGE // 16):
                        rows_v[b][j, pl.ds(k2 * 16, 16)] = (
                            rows_v[b][j, pl.ds(k2 * 16, 16)] * w)
                return c2

            lax.fori_loop(0, EC // 16, rowgrp, 0)
            for g in range(EC // 16):
                dst_s[b][pl.ds(g * 16, 16)] = dst_c[b][pl.ds(g * 16, 16)]
            pltpu.async_copy(rows_v[b], u_sh.at[dst_s[b]], sem_w[b],
                             add=True)
            pltpu.async_copy(ex_v[b], s_sh.at[dst_s[b]], sem_w[b],
                             add=True)

        def body(i, p):
            p2 = (p + 2) % 4
            p3 = (p + 3) % 4

            @pl.when(i + 3 < NCHUNK)
            def _():
                fire_idx(i + 3, p3)

            @pl.when(i + 2 < NCHUNK)
            def _():
                @pl.when(i >= 2)
                def _():
                    wait_scat(p2)

                wait_idx(p2)
                fire_gather(p2)

            process(p)

        fire_idx(0, 0, sync=True)
        fire_idx(1, 1, sync=True)
        fire_gather(0)
        fire_gather(1)
        fire_idx(2, 2)

        def quad(i4, carry):
            for b in range(4):
                body(i4 * 4 + b, b)
            return carry

        lax.fori_loop(0, NCHUNK // 4, quad, 0)
        body(NCHUNK - 1, 0)
        wait_scat(2)
        wait_scat(3)
        wait_scat(0)
        plsc.subcore_barrier()
        pltpu.sync_copy(u_sh.at[pl.ds(stripe, STRIPE)],
                        u_out.at[cid, pl.ds(stripe, STRIPE)])
        pltpu.sync_copy(s_sh.at[pl.ds(stripe, STRIPE)],
                        s_out.at[cid, pl.ds(stripe, STRIPE)])

    return k(h, asrc, adst, src, dst, zrows, zs)


def _tc_pre(x, W, A):
    def body(x_ref, w_ref, a_ref, h_ref, av_ref):
        hv = jnp.dot(x_ref[...], w_ref[...], preferred_element_type=jnp.float32)
        h_ref[...] = hv
        av_ref[...] = jnp.dot(hv, a_ref[...], preferred_element_type=jnp.float32)

    return pl.pallas_call(
        body,
        out_shape=(
            jax.ShapeDtypeStruct((NPAD, KGE), jnp.float32),
            jax.ShapeDtypeStruct((NPAD, KGE), jnp.float32),
        ),
    )(x, W, A)


def _tc_mid(u, s3, h, av, b_row, W2, A2):
    def body(u_ref, s_ref, h_ref, av_ref, b_ref, w_ref, a_ref, h2_ref, av2_ref):
        exl = jnp.exp(jnp.where(av_ref[:, 0:1] > 0.0, av_ref[:, 0:1],
                                0.2 * av_ref[:, 0:1]))
        stot = s_ref[0] + s_ref[1] + exl + 1e-16
        x2 = (u_ref[0] + u_ref[1] + exl * h_ref[...]) / stot + b_ref[...]
        x2 = jnp.maximum(x2, 0.0)
        h2 = jnp.dot(x2, w_ref[...], preferred_element_type=jnp.float32)
        h2_ref[...] = h2
        av2_ref[...] = jnp.dot(h2, a_ref[...], preferred_element_type=jnp.float32)

    return pl.pallas_call(
        body,
        out_shape=(
            jax.ShapeDtypeStruct((NPAD, KGE), jnp.float32),
            jax.ShapeDtypeStruct((NPAD, KGE), jnp.float32),
        ),
    )(u, s3, h, av, b_row, W2, A2)


def _tc_post(u, s3, h, av, b_row, Wc, bc_row):
    def body(u_ref, s_ref, h_ref, av_ref, b_ref, wc_ref, bc_ref, hc_ref):
        exl = jnp.exp(jnp.where(av_ref[:, 0:1] > 0.0, av_ref[:, 0:1],
                                0.2 * av_ref[:, 0:1]))
        stot = s_ref[0] + s_ref[1] + exl + 1e-16
        out2 = (u_ref[0] + u_ref[1] + exl * h_ref[...]) / stot + b_ref[...]
        hc_ref[...] = jnp.dot(out2, wc_ref[...],
                              preferred_element_type=jnp.float32) + bc_ref[...]

    return pl.pallas_call(
        body,
        out_shape=jax.ShapeDtypeStruct((NPAD, KGE), jnp.float32),
    )(u, s3, h, av, b_row, Wc, bc_row)


def _tc_pool(hc, batch2d, hidden, seq_lengths, Wd, bd_row, Wu, bu_row):
    def body(hc_ref, b2d_ref, hid_ref, sl_ref, wd_ref, bd_ref, wu_ref,
             bu_ref, out_ref):
        i = pl.program_id(0)
        b2d = b2d_ref[...]
        c = jnp.sum(jnp.where(b2d == i, 1, 0))
        off = jnp.sum(jnp.where(b2d < i, 1, 0))
        sl = sl_ref[i]
        L = c + sl
        p2 = lax.broadcasted_iota(jnp.int32, (P, 1), 0)
        s_ = (p2 * L) // P
        e_ = ((p2 + 1) * L + (P - 1)) // P
        lo_n = off + s_
        hi_n = off + jnp.minimum(e_, c)
        iota_n = lax.broadcasted_iota(jnp.int32, (P, NPAD), 1)
        mask_n = ((iota_n >= lo_n) & (iota_n < hi_n)).astype(jnp.float32)
        s_hc = jnp.dot(mask_n, hc_ref[...], preferred_element_type=jnp.float32)
        lo_h = jnp.maximum(s_, c) - c
        hi_h = jnp.maximum(e_ - c, 0)
        iota_s = lax.broadcasted_iota(jnp.int32, (P, S), 1)
        mask_h = ((iota_s >= lo_h) & (iota_s < hi_h)).astype(jnp.float32)
        s_raw = jnp.dot(mask_h, hid_ref[0], preferred_element_type=jnp.float32)
        s_hid = jnp.dot(s_raw, wd_ref[...], preferred_element_type=jnp.float32)
        nh = (hi_h - lo_h).astype(jnp.float32)
        w = (e_ - s_).astype(jnp.float32)
        pool = (s_hc + s_hid + nh * bd_ref[...]) / w
        silu = pool * (1.0 / (1.0 + jnp.exp(-pool)))
        out_ref[0] = jnp.dot(silu, wu_ref[...],
                             preferred_element_type=jnp.float32) + bu_ref[...]

    return pl.pallas_call(
        body,
        grid=(B,),
        in_specs=[
            pl.BlockSpec((NPAD, KGE), lambda i: (0, 0)),
            pl.BlockSpec((NPAD // 128, 128), lambda i: (0, 0)),
            pl.BlockSpec((1, S, H), lambda i: (i, 0, 0)),
            pl.BlockSpec(memory_space=pltpu.SMEM),
            pl.BlockSpec((H, D), lambda i: (0, 0)),
            pl.BlockSpec((1, D), lambda i: (0, 0)),
            pl.BlockSpec((D, H), lambda i: (0, 0)),
            pl.BlockSpec((1, H), lambda i: (0, 0)),
        ],
        out_specs=pl.BlockSpec((1, P, H), lambda i: (i, 0, 0)),
        out_shape=jax.ShapeDtypeStruct((B, P, H), jnp.float32),
    )(hc, batch2d, hidden, seq_lengths, Wd, bd_row, Wu, bu_row)


def kernel(graph_x, edge_index, batch, hidden_states, seq_lengths, emb,
           W1, as1, ad1, b1, W2, as2, ad2, b2, Wd, bd, Wc, bc, Wu, bu):
    f32 = jnp.float32
    gx_pad = jnp.pad(graph_x, (0, NPAD - N))
    src = edge_index[0]
    dst = edge_index[1]
    batch2d = jnp.pad(batch, (0, NPAD - N), constant_values=127).reshape(
        NPAD // 128, 128)
    zrows = jnp.zeros((STRIPE, KGE), f32)
    zs = jnp.zeros((STRIPE,), f32)
    zcol = jnp.zeros((KGE, KGE - 3), f32)
    A1 = jnp.concatenate(
        [(as1 + ad1)[:, None], as1[:, None], ad1[:, None], zcol], axis=1)
    A2 = jnp.concatenate(
        [(as2 + ad2)[:, None], as2[:, None], ad2[:, None], zcol], axis=1)

    gx = _emb_gather(emb, gx_pad)

    h1, av1 = _tc_pre(gx, W1, A1)
    u1, s1 = _edge_pass(h1, av1[:, 1], av1[:, 2], src, dst, zrows, zs)
    h2, av2 = _tc_mid(u1, s1.reshape(2, NPAD, 1), h1, av1,
                      b1.reshape(1, KGE), W2, A2)
    u2, s2 = _edge_pass(h2, av2[:, 1], av2[:, 2], src, dst, zrows, zs)
    hc = _tc_post(u2, s2.reshape(2, NPAD, 1), h2, av2,
                  b2.reshape(1, KGE), Wc, bc.reshape(1, KGE))

    return _tc_pool(hc, batch2d, hidden_states, seq_lengths,
                    Wd, bd.reshape(1, D), Wu, bu.reshape(1, H))
